# baseline (speedup 1.0000x reference)
import functools

import jax
import jax.numpy as jnp
from jax import lax
from jax.experimental import pallas as pl
from jax.experimental.pallas import tpu as pltpu
from jax.experimental.pallas import tpu_sc as plsc

S = 16384
E = 64
NUM_ROWS = 1000

NC = 2
NS = 16
NW = NC * NS
TOK_PER_W = S // NW
CHUNK = 128
NCH = TOK_PER_W // CHUNK
OUT_W = 128
FIN_B = 1024


def _project_body(ent_ref, rel_ref, w_ref, b_ref, tbl_ref):
    ent = ent_ref[...]
    rel = rel_ref[...]
    w = w_ref[...]
    dn = (((1,), (1,)), ((), ()))
    tbl_ref[0:NUM_ROWS, :] = lax.dot_general(
        ent, w[:, 0:E], dn, preferred_element_type=jnp.float32)
    tbl_ref[NUM_ROWS:2 * NUM_ROWS, :] = lax.dot_general(
        rel, w[:, E:2 * E], dn, preferred_element_type=jnp.float32) + b_ref[...]
    tbl_ref[2 * NUM_ROWS:3 * NUM_ROWS, :] = lax.dot_general(
        ent, w[:, 2 * E:3 * E], dn, preferred_element_type=jnp.float32)


def _project_tables(entity_emb, relation_emb, W_tok, b_tok):
    return pl.pallas_call(
        _project_body,
        out_shape=jax.ShapeDtypeStruct((3 * NUM_ROWS, E), jnp.float32),
    )(entity_emb, relation_emb, W_tok, b_tok.reshape(1, E))


def _sc_body(ms_hbm, tbl_hbm, out_hbm, msb, ih, ir, it, gh, gr, gt,
             semg0, semg1, semw):
    wid = lax.axis_index("s") * NC + lax.axis_index("c")
    base = pl.multiple_of(wid * TOK_PER_W, TOK_PER_W)
    row0 = pl.multiple_of(
        (wid // 4) * FIN_B + (wid % 2) * TOK_PER_W, TOK_PER_W)
    col0 = ((wid % 4) // 2) * E

    pltpu.sync_copy(ms_hbm.at[pl.ds(base, TOK_PER_W), :], msb)

    lane = lax.iota(jnp.int32, 16)
    for g in range(TOK_PER_W // 16):
        rows = lane + g * 16
        c, pos = g // (CHUNK // 16), (g % (CHUNK // 16)) * 16
        sl = pl.ds(pos, 16)
        ih[c, sl] = plsc.load_gather(msb, [rows, jnp.zeros(16, jnp.int32)])
        ir[c, sl] = plsc.load_gather(
            msb, [rows, jnp.ones(16, jnp.int32)]) + NUM_ROWS
        it[c, sl] = plsc.load_gather(
            msb, [rows, jnp.full(16, 2, jnp.int32)]) + 2 * NUM_ROWS

    def start_gathers(c):
        b = c % 2
        sem = semg0 if b == 0 else semg1
        return (pltpu.async_copy(tbl_hbm.at[ih.at[c]], gh.at[b], sem),
                pltpu.async_copy(tbl_hbm.at[ir.at[c]], gr.at[b], sem),
                pltpu.async_copy(tbl_hbm.at[it.at[c]], gt.at[b], sem))

    def compute(c):
        b = c % 2

        def body(i, carry):
            for j in range(E // 16):
                sl = pl.ds(j * 16, 16)
                plsc.addupdate(gh.at[b, i, sl], gr[b, i, sl] + gt[b, i, sl])
            return carry

        lax.fori_loop(0, CHUNK, body, 0)

    gcur = start_gathers(0)
    wbs = {}
    for c in range(NCH):
        b = c % 2
        if c + 1 < NCH:
            if c - 1 >= 0:
                wbs.pop(c - 1).wait()
            gnext = start_gathers(c + 1)
        for d in gcur:
            d.wait()
        compute(c)
        wbs[c] = pltpu.async_copy(
            gh.at[b],
            out_hbm.at[pl.ds(row0 + c * CHUNK, CHUNK), pl.ds(col0, E)],
            semw)
        if c + 1 < NCH:
            gcur = gnext
    for c in sorted(wbs):
        wbs[c].wait()


_sc_gather = functools.partial(
    pl.kernel,
    out_type=jax.ShapeDtypeStruct((S * E // OUT_W, OUT_W), jnp.float32),
    mesh=plsc.VectorSubcoreMesh(core_axis_name="c", subcore_axis_name="s"),
    scratch_types=[
        pltpu.VMEM((TOK_PER_W, 3), jnp.int32),
        pltpu.VMEM((NCH, CHUNK), jnp.int32),
        pltpu.VMEM((NCH, CHUNK), jnp.int32),
        pltpu.VMEM((NCH, CHUNK), jnp.int32),
        pltpu.VMEM((2, CHUNK, E), jnp.float32),
        pltpu.VMEM((2, CHUNK, E), jnp.float32),
        pltpu.VMEM((2, CHUNK, E), jnp.float32),
        pltpu.SemaphoreType.DMA,
        pltpu.SemaphoreType.DMA,
        pltpu.SemaphoreType.DMA,
    ],
    compiler_params=pltpu.CompilerParams(use_tc_tiling_on_sc=False,
                                         needs_layout_passes=False),
)(_sc_body)


def _finish_body(in_ref, out_ref):
    out_ref[0:FIN_B, :] = in_ref[:, 0:E]
    out_ref[FIN_B:2 * FIN_B, :] = in_ref[:, E:OUT_W]


def _finish(packed):
    grid = (S * E // OUT_W) // FIN_B
    return pl.pallas_call(
        _finish_body,
        grid=(grid,),
        in_specs=[pl.BlockSpec((FIN_B, OUT_W), lambda g: (g, 0))],
        out_specs=pl.BlockSpec((2 * FIN_B, E), lambda g: (g, 0)),
        out_shape=jax.ShapeDtypeStruct((S, E), jnp.float32),
    )(packed)


def kernel(memory_state, entity_emb, relation_emb, W_tok, b_tok):
    tbl = _project_tables(entity_emb, relation_emb, W_tok, b_tok)
    packed = _sc_gather(memory_state, tbl)
    return _finish(packed)


# speedup vs baseline: 1.6087x; 1.6087x over previous
import functools

import jax
import jax.numpy as jnp
from jax import lax
from jax.experimental import pallas as pl
from jax.experimental.pallas import tpu as pltpu
from jax.experimental.pallas import tpu_sc as plsc

S = 16384
E = 64
NUM_ROWS = 1000

NC = 2
NS = 16
NW = NC * NS
TOK_PER_W = S // NW
CHUNK = 128
NCH = TOK_PER_W // CHUNK
OUT_W = 128
FIN_B = 1024


def _project_body(entT_ref, relT_ref, w_ref, b_ref, tbl_ref):
    entT = entT_ref[...]
    relT = relT_ref[...]
    w = w_ref[...]
    dn = (((0,), (1,)), ((), ()))
    tbl_ref[0:NUM_ROWS, :] = lax.dot_general(
        entT, w[:, 0:E], dn, preferred_element_type=jnp.float32)
    tbl_ref[NUM_ROWS:2 * NUM_ROWS, :] = lax.dot_general(
        relT, w[:, E:2 * E], dn, preferred_element_type=jnp.float32) + b_ref[...]
    tbl_ref[2 * NUM_ROWS:3 * NUM_ROWS, :] = lax.dot_general(
        entT, w[:, 2 * E:3 * E], dn, preferred_element_type=jnp.float32)


def _project_tables(entity_emb, relation_emb, W_tok, b_tok):
    return pl.pallas_call(
        _project_body,
        out_shape=jax.ShapeDtypeStruct((3 * NUM_ROWS, E), jnp.float32),
    )(entity_emb.T, relation_emb.T, W_tok, b_tok.reshape(1, E))


def _sc_body(msT_hbm, tbl_hbm, out_hbm, hbuf, rbuf, tbuf, gh, gr, gt,
             semg0, semg1, semw):
    wid = lax.axis_index("s") * NC + lax.axis_index("c")
    base = pl.multiple_of(wid * TOK_PER_W, TOK_PER_W)
    row0 = pl.multiple_of(
        (wid // 4) * FIN_B + (wid % 2) * TOK_PER_W, TOK_PER_W)
    col0 = ((wid % 4) // 2) * E

    pltpu.sync_copy(msT_hbm.at[0, pl.ds(base, TOK_PER_W)], hbuf)
    pltpu.sync_copy(msT_hbm.at[1, pl.ds(base, TOK_PER_W)], rbuf)
    pltpu.sync_copy(msT_hbm.at[2, pl.ds(base, TOK_PER_W)], tbuf)

    for g in range(TOK_PER_W // 16):
        sl = pl.ds(g * 16, 16)
        rbuf[sl] = rbuf[sl] + NUM_ROWS
        tbuf[sl] = tbuf[sl] + 2 * NUM_ROWS

    def start_gathers(c):
        b = c % 2
        sem = semg0 if b == 0 else semg1
        sl = pl.ds(c * CHUNK, CHUNK)
        return (pltpu.async_copy(tbl_hbm.at[hbuf.at[sl]], gh.at[b], sem),
                pltpu.async_copy(tbl_hbm.at[rbuf.at[sl]], gr.at[b], sem),
                pltpu.async_copy(tbl_hbm.at[tbuf.at[sl]], gt.at[b], sem))

    def compute(c):
        b = c % 2

        def body(i, carry):
            for j in range(E // 16):
                sl = pl.ds(j * 16, 16)
                plsc.addupdate(gh.at[b, i, sl], gr[b, i, sl] + gt[b, i, sl])
            return carry

        lax.fori_loop(0, CHUNK, body, 0)

    gcur = start_gathers(0)
    wbs = {}
    for c in range(NCH):
        b = c % 2
        if c + 1 < NCH:
            if c - 1 >= 0:
                wbs.pop(c - 1).wait()
            gnext = start_gathers(c + 1)
        for d in gcur:
            d.wait()
        compute(c)
        wbs[c] = pltpu.async_copy(
            gh.at[b],
            out_hbm.at[pl.ds(row0 + c * CHUNK, CHUNK), pl.ds(col0, E)],
            semw)
        if c + 1 < NCH:
            gcur = gnext
    for c in sorted(wbs):
        wbs[c].wait()


_sc_gather = functools.partial(
    pl.kernel,
    out_type=jax.ShapeDtypeStruct((S * E // OUT_W, OUT_W), jnp.float32),
    mesh=plsc.VectorSubcoreMesh(core_axis_name="c", subcore_axis_name="s"),
    scratch_types=[
        pltpu.VMEM((TOK_PER_W,), jnp.int32),
        pltpu.VMEM((TOK_PER_W,), jnp.int32),
        pltpu.VMEM((TOK_PER_W,), jnp.int32),
        pltpu.VMEM((2, CHUNK, E), jnp.float32),
        pltpu.VMEM((2, CHUNK, E), jnp.float32),
        pltpu.VMEM((2, CHUNK, E), jnp.float32),
        pltpu.SemaphoreType.DMA,
        pltpu.SemaphoreType.DMA,
        pltpu.SemaphoreType.DMA,
    ],
    compiler_params=pltpu.CompilerParams(use_tc_tiling_on_sc=False,
                                         needs_layout_passes=False),
)(_sc_body)


def _finish_body(in_ref, out_ref):
    x = in_ref[...]
    out_ref[:, 0:FIN_B] = x[:, 0:E].T
    out_ref[:, FIN_B:2 * FIN_B] = x[:, E:OUT_W].T


def _finish(packed):
    grid = (S * E // OUT_W) // FIN_B
    return pl.pallas_call(
        _finish_body,
        grid=(grid,),
        in_specs=[pl.BlockSpec((FIN_B, OUT_W), lambda g: (g, 0))],
        out_specs=pl.BlockSpec((E, 2 * FIN_B), lambda g: (0, g)),
        out_shape=jax.ShapeDtypeStruct((E, S), jnp.float32),
    )(packed)


def kernel(memory_state, entity_emb, relation_emb, W_tok, b_tok):
    tbl = _project_tables(entity_emb, relation_emb, W_tok, b_tok)
    packed = _sc_gather(memory_state.T, tbl)
    return _finish(packed).T
